# initial kernel scaffold (unmeasured)
import jax
import jax.numpy as jnp
from jax import lax
from jax.experimental import pallas as pl
from jax.experimental.pallas import tpu as pltpu


def kernel(
    x,
):
    def body(*refs):
        pass

    out_shape = jax.ShapeDtypeStruct(..., jnp.float32)
    return pl.pallas_call(body, out_shape=out_shape)(...)


# baseline (device time: 81386 ns/iter reference)
import jax
import jax.numpy as jnp
from jax import lax
from jax.experimental import pallas as pl
from jax.experimental.pallas import tpu as pltpu

N_DEV = 16
N_CHUNK = 16


def kernel(x):
    m, n = x.shape
    rows = m // N_CHUNK

    def body(x_ref, out_ref, comm_ref, send_sems, recv_sems):
        my = lax.axis_index("i")
        left = lax.rem(my - 1 + N_DEV, N_DEV)
        right = lax.rem(my + 1, N_DEV)

        barrier_sem = pltpu.get_barrier_semaphore()
        for nbr in (left, right):
            pl.semaphore_signal(
                barrier_sem, inc=1,
                device_id=(nbr,), device_id_type=pl.DeviceIdType.MESH,
            )
        pl.semaphore_wait(barrier_sem, 2)

        out_ref[...] = x_ref[...]

        def cidx(c):
            return lax.rem(c + 8 * N_CHUNK, N_CHUNK) * rows

        def chunk(ref, c):
            return ref.at[pl.ds(cidx(c), rows), :]

        for s in range(N_DEV - 1):
            rdma = pltpu.make_async_remote_copy(
                src_ref=chunk(out_ref, my - s),
                dst_ref=comm_ref.at[s],
                send_sem=send_sems.at[s],
                recv_sem=recv_sems.at[s],
                device_id=(right,),
                device_id_type=pl.DeviceIdType.MESH,
            )
            rdma.start()
            rdma.wait()
            i = cidx(my - s - 1)
            out_ref[pl.ds(i, rows), :] = out_ref[pl.ds(i, rows), :] + comm_ref[s]

        for s in range(N_DEV - 1):
            rdma = pltpu.make_async_remote_copy(
                src_ref=chunk(out_ref, my + 1 - s),
                dst_ref=chunk(out_ref, my + 1 - s),
                send_sem=send_sems.at[N_DEV - 1 + s],
                recv_sem=recv_sems.at[N_DEV - 1 + s],
                device_id=(right,),
                device_id_type=pl.DeviceIdType.MESH,
            )
            rdma.start()
            rdma.wait()

    n_sems = 2 * (N_DEV - 1)
    return pl.pallas_call(
        body,
        out_shape=jax.ShapeDtypeStruct((m, n), x.dtype),
        in_specs=[pl.BlockSpec(memory_space=pltpu.VMEM)],
        out_specs=pl.BlockSpec(memory_space=pltpu.VMEM),
        scratch_shapes=[
            pltpu.VMEM((N_DEV - 1, rows, n), x.dtype),
            pltpu.SemaphoreType.DMA((n_sems,)),
            pltpu.SemaphoreType.DMA((n_sems,)),
        ],
        compiler_params=pltpu.CompilerParams(collective_id=0),
    )(x)


# device time: 38736 ns/iter; 2.1010x vs baseline; 2.1010x over previous
import jax
import jax.numpy as jnp
from jax import lax
from jax.experimental import pallas as pl
from jax.experimental.pallas import tpu as pltpu

N_DEV = 16
MASKS = (1, 3, 4, 8)
HALVES = (256, 128, 64, 32)


def kernel(x):
    m, n = x.shape

    def body(x_ref, out_ref, c0, c1, c2, c3, send_sems, recv_sems):
        my = lax.axis_index("i")
        b0 = my & 1
        b1 = (my >> 1) & 1
        b2 = (my >> 2) & 1
        b3 = (my >> 3) & 1
        sides = (b0 ^ b1, b1, b2, b3)
        partners = tuple(my ^ mk for mk in MASKS)
        comms = (c0, c1, c2, c3)

        barrier_sem = pltpu.get_barrier_semaphore()
        for p in partners:
            pl.semaphore_signal(
                barrier_sem, inc=1,
                device_id=(p,), device_id_type=pl.DeviceIdType.MESH,
            )
        pl.semaphore_wait(barrier_sem, len(MASKS))

        out_ref[...] = x_ref[...]

        start = my * 0
        for k in range(4):
            half = HALVES[k]
            send_start = start + (1 - sides[k]) * half
            keep_start = start + sides[k] * half
            rdma = pltpu.make_async_remote_copy(
                src_ref=out_ref.at[pl.ds(send_start, half), :],
                dst_ref=comms[k],
                send_sem=send_sems.at[k],
                recv_sem=recv_sems.at[k],
                device_id=(partners[k],),
                device_id_type=pl.DeviceIdType.MESH,
            )
            rdma.start()
            rdma.wait()
            out_ref[pl.ds(keep_start, half), :] = (
                out_ref[pl.ds(keep_start, half), :] + comms[k][...]
            )
            start = keep_start

        size = HALVES[3]
        for j, k in enumerate(reversed(range(4))):
            rdma = pltpu.make_async_remote_copy(
                src_ref=out_ref.at[pl.ds(start, size), :],
                dst_ref=out_ref.at[pl.ds(start, size), :],
                send_sem=send_sems.at[4 + j],
                recv_sem=recv_sems.at[4 + j],
                device_id=(partners[k],),
                device_id_type=pl.DeviceIdType.MESH,
            )
            rdma.start()
            rdma.wait()
            start = start - sides[k] * size
            size *= 2

    return pl.pallas_call(
        body,
        out_shape=jax.ShapeDtypeStruct((m, n), x.dtype),
        in_specs=[pl.BlockSpec(memory_space=pltpu.VMEM)],
        out_specs=pl.BlockSpec(memory_space=pltpu.VMEM),
        scratch_shapes=[
            pltpu.VMEM((HALVES[0], n), x.dtype),
            pltpu.VMEM((HALVES[1], n), x.dtype),
            pltpu.VMEM((HALVES[2], n), x.dtype),
            pltpu.VMEM((HALVES[3], n), x.dtype),
            pltpu.SemaphoreType.DMA((8,)),
            pltpu.SemaphoreType.DMA((8,)),
        ],
        compiler_params=pltpu.CompilerParams(collective_id=0),
    )(x)


# device time: 33894 ns/iter; 2.4012x vs baseline; 1.1429x over previous
import jax
import jax.numpy as jnp
from jax import lax
from jax.experimental import pallas as pl
from jax.experimental.pallas import tpu as pltpu

N_DEV = 16
ORDERS = ((1, 3, 4, 8), (3, 1, 8, 4))
HALVES = (256, 128, 64, 32)
N_STREAM = 2


def kernel(x):
    m, n = x.shape
    nh = n // N_STREAM

    def body(x_ref, out_ref, *scratch):
        comms = scratch[: 4 * N_STREAM]
        send_sems, recv_sems = scratch[4 * N_STREAM :]

        my = lax.axis_index("i")
        b0 = my & 1
        b1 = (my >> 1) & 1
        b2 = (my >> 2) & 1
        b3 = (my >> 3) & 1
        side_of = {1: b0 ^ b1, 3: b1, 4: b2, 8: b3}

        barrier_sem = pltpu.get_barrier_semaphore()
        for mk in (1, 3, 4, 8):
            pl.semaphore_signal(
                barrier_sem, inc=1,
                device_id=(my ^ mk,), device_id_type=pl.DeviceIdType.MESH,
            )
        pl.semaphore_wait(barrier_sem, 4)

        out_ref[...] = x_ref[...]

        def col(h):
            return pl.ds(h * nh, nh)

        starts = [my * 0 for _ in range(N_STREAM)]
        for k in range(4):
            half = HALVES[k]
            rdmas = []
            for h in range(N_STREAM):
                mk = ORDERS[h][k]
                a = side_of[mk]
                send_start = starts[h] + (1 - a) * half
                rdma = pltpu.make_async_remote_copy(
                    src_ref=out_ref.at[pl.ds(send_start, half), col(h)],
                    dst_ref=comms[4 * h + k],
                    send_sem=send_sems.at[N_STREAM * k + h],
                    recv_sem=recv_sems.at[N_STREAM * k + h],
                    device_id=(my ^ mk,),
                    device_id_type=pl.DeviceIdType.MESH,
                )
                rdma.start()
                rdmas.append(rdma)
            for h in range(N_STREAM):
                a = side_of[ORDERS[h][k]]
                keep = starts[h] + a * half
                rdmas[h].wait()
                out_ref[pl.ds(keep, half), col(h)] = (
                    out_ref[pl.ds(keep, half), col(h)] + comms[4 * h + k][...]
                )
                starts[h] = keep

        for j, k in enumerate(reversed(range(4))):
            size = HALVES[k]
            rdmas = []
            for h in range(N_STREAM):
                mk = ORDERS[h][k]
                rdma = pltpu.make_async_remote_copy(
                    src_ref=out_ref.at[pl.ds(starts[h], size), col(h)],
                    dst_ref=out_ref.at[pl.ds(starts[h], size), col(h)],
                    send_sem=send_sems.at[8 + N_STREAM * j + h],
                    recv_sem=recv_sems.at[8 + N_STREAM * j + h],
                    device_id=(my ^ mk,),
                    device_id_type=pl.DeviceIdType.MESH,
                )
                rdma.start()
                rdmas.append(rdma)
            for h in range(N_STREAM):
                rdmas[h].wait()
                starts[h] = starts[h] - side_of[ORDERS[h][k]] * size

    n_sems = 8 * N_STREAM
    comm_shapes = [
        pltpu.VMEM((HALVES[k], nh), x.dtype)
        for _ in range(N_STREAM)
        for k in range(4)
    ]
    return pl.pallas_call(
        body,
        out_shape=jax.ShapeDtypeStruct((m, n), x.dtype),
        in_specs=[pl.BlockSpec(memory_space=pltpu.VMEM)],
        out_specs=pl.BlockSpec(memory_space=pltpu.VMEM),
        scratch_shapes=comm_shapes
        + [
            pltpu.SemaphoreType.DMA((n_sems,)),
            pltpu.SemaphoreType.DMA((n_sems,)),
        ],
        compiler_params=pltpu.CompilerParams(collective_id=0),
    )(x)
